# baseline (speedup 1.0000x reference)
import functools

import jax
import jax.numpy as jnp
from jax.experimental import pallas as pl
from jax.experimental.pallas import tpu as pltpu
from jax.experimental.pallas import tpu_sc as plsc

_GC = 128
_WORKERS = 32
_NB = 200
_NCHUNKS = 5


def _gelu(x):
    return x * 0.5 * (1.0 + jax.lax.erf(x * 0.7071067811865476))


def _ln(x, g, b):
    m = jnp.mean(x, axis=-1, keepdims=True)
    d = x - m
    v = jnp.mean(d * d, axis=-1, keepdims=True)
    return d * jax.lax.rsqrt(v + 1e-5) * g + b


def _sc_gather(table, idx1d):
    Mp = idx1d.shape[0]
    D = table.shape[1]
    idx2d = idx1d.reshape(1, Mp)
    mesh = plsc.VectorSubcoreMesh(core_axis_name="c", subcore_axis_name="s")

    @functools.partial(
        pl.kernel,
        out_type=jax.ShapeDtypeStruct((Mp, D), table.dtype),
        mesh=mesh,
    )
    def run(x_hbm, i_hbm, o_hbm):
        def body(i_vmem, o_vmem):
            pltpu.sync_copy(x_hbm.at[i_vmem.at[0]], o_vmem)

        pltpu.emit_pipeline(
            body,
            grid=(Mp // _GC,),
            in_specs=[pl.BlockSpec((1, _GC), lambda i: (0, i))],
            out_specs=[pl.BlockSpec((_GC, D), lambda i: (i, 0))],
            core_axis_name=("c", "s"),
            dimension_semantics=(pltpu.PARALLEL,),
        )(i_hbm, o_hbm)

    return run(table, idx2d)


def _proj_body(hv, w1a, b1, w1c, pre1_o, g1_o):
    x = hv[...]
    pre1_o[...] = (
        jnp.dot(x, w1a[...], preferred_element_type=jnp.float32) + b1[...]
    )
    g1_o[...] = jnp.dot(x, w1c[...], preferred_element_type=jnp.float32)


def _block1_body(hv, pre1, he, g1, w1b, w2, b2, w3, b3, win, bi, wout, bo,
                 n1g, n1b, n2g, n2b, w11a, b11, w11c,
                 hv2_o, pre11_o, g11_o):
    nb, Hd = hv.shape
    ne = he.shape[0]
    K = ne // nb
    e = jnp.dot(he[...], w1b[...], preferred_element_type=jnp.float32)
    t = (e + g1[...]).reshape(nb, K, Hd) + pre1[...][:, None, :]
    t = _gelu(t).reshape(ne, Hd)
    u = _gelu(jnp.dot(t, w2[...], preferred_element_type=jnp.float32) + b2[...])
    msg = jnp.dot(u, w3[...], preferred_element_type=jnp.float32) + b3[...]
    dh = msg.reshape(nb, K, Hd).sum(axis=1) * (1.0 / 30.0)
    h = _ln(hv[...] + dh, n1g[...], n1b[...])
    f = _gelu(jnp.dot(h, win[...], preferred_element_type=jnp.float32) + bi[...])
    f = jnp.dot(f, wout[...], preferred_element_type=jnp.float32) + bo[...]
    y = _ln(h + f, n2g[...], n2b[...])
    hv2_o[...] = y
    pre11_o[...] = (
        jnp.dot(y, w11a[...], preferred_element_type=jnp.float32) + b11[...]
    )
    g11_o[...] = jnp.dot(y, w11c[...], preferred_element_type=jnp.float32)


def _block2_body(he, g2, pre11, w11b, w12, b12, w13, b13, n3g, n3b, heo):
    ne, Hd = he.shape
    nb = pre11.shape[0]
    K = ne // nb
    e = jnp.dot(he[...], w11b[...], preferred_element_type=jnp.float32)
    t = (e + g2[...]).reshape(nb, K, Hd) + pre11[...][:, None, :]
    t = _gelu(t).reshape(ne, Hd)
    u = _gelu(jnp.dot(t, w12[...], preferred_element_type=jnp.float32) + b12[...])
    msg = jnp.dot(u, w13[...], preferred_element_type=jnp.float32) + b13[...]
    heo[...] = _ln(he[...] + msg, n3g[...], n3b[...])


def _row_spec(rows, cols):
    return pl.BlockSpec((rows, cols), lambda i: (i, 0))


def _const_spec(rows, cols):
    return pl.BlockSpec((rows, cols), lambda i: (0, 0))


def kernel(h_V, h_E, E_idx, params):
    B, N, Hd = h_V.shape
    K = h_E.shape[2]
    M = N * K
    hv = h_V.reshape(N, Hd)
    he = h_E.reshape(M, Hd)
    idx = E_idx.reshape(-1).astype(jnp.int32)

    p = params
    w1 = p["W1"]["w"]
    w11 = p["W11"]["w"]
    w1a, w1b, w1c = w1[:Hd], w1[Hd:2 * Hd], w1[2 * Hd:]
    w11a, w11b, w11c = w11[:Hd], w11[Hd:2 * Hd], w11[2 * Hd:]
    b1 = p["W1"]["b"].reshape(1, Hd)
    b11 = p["W11"]["b"].reshape(1, Hd)
    b2 = p["W2"]["b"].reshape(1, Hd)
    b3 = p["W3"]["b"].reshape(1, Hd)
    b12 = p["W12"]["b"].reshape(1, Hd)
    b13 = p["W13"]["b"].reshape(1, Hd)
    bi = p["Win"]["b"].reshape(1, -1)
    bo = p["Wout"]["b"].reshape(1, Hd)
    n1g = p["norm1"]["g"].reshape(1, Hd)
    n1b = p["norm1"]["b"].reshape(1, Hd)
    n2g = p["norm2"]["g"].reshape(1, Hd)
    n2b = p["norm2"]["b"].reshape(1, Hd)
    n3g = p["norm3"]["g"].reshape(1, Hd)
    n3b = p["norm3"]["b"].reshape(1, Hd)
    Hi = p["Win"]["w"].shape[1]

    nchunk = N // _NCHUNKS
    echunk = nchunk * K
    align = _GC * _WORKERS
    epad = ((echunk + align - 1) // align) * align
    zpad = jnp.zeros((epad - echunk,), jnp.int32) if epad != echunk else None

    def idx_chunk(c):
        ic = idx[c * echunk:(c + 1) * echunk]
        if zpad is not None:
            ic = jnp.concatenate([ic, zpad])
        return ic

    pre1, g1t = pl.pallas_call(
        _proj_body,
        out_shape=[
            jax.ShapeDtypeStruct((N, Hd), jnp.float32),
            jax.ShapeDtypeStruct((N, Hd), jnp.float32),
        ],
    )(hv, w1a, b1, w1c)

    grid = (nchunk // _NB,)
    ne = _NB * K

    def call_block1(hv_c, pre1_c, he_c, g1_c):
        return pl.pallas_call(
            _block1_body,
            grid=grid,
            in_specs=[
                _row_spec(_NB, Hd),
                _row_spec(_NB, Hd),
                _row_spec(ne, Hd),
                _row_spec(ne, Hd),
                _const_spec(Hd, Hd),
                _const_spec(Hd, Hd),
                _const_spec(1, Hd),
                _const_spec(Hd, Hd),
                _const_spec(1, Hd),
                _const_spec(Hd, Hi),
                _const_spec(1, Hi),
                _const_spec(Hi, Hd),
                _const_spec(1, Hd),
                _const_spec(1, Hd),
                _const_spec(1, Hd),
                _const_spec(1, Hd),
                _const_spec(1, Hd),
                _const_spec(Hd, Hd),
                _const_spec(1, Hd),
                _const_spec(Hd, Hd),
            ],
            out_specs=[
                _row_spec(_NB, Hd),
                _row_spec(_NB, Hd),
                _row_spec(_NB, Hd),
            ],
            out_shape=[
                jax.ShapeDtypeStruct((nchunk, Hd), jnp.float32),
                jax.ShapeDtypeStruct((nchunk, Hd), jnp.float32),
                jax.ShapeDtypeStruct((nchunk, Hd), jnp.float32),
            ],
        )(hv_c, pre1_c, he_c, g1_c, w1b, p["W2"]["w"], b2, p["W3"]["w"], b3,
          p["Win"]["w"], bi, p["Wout"]["w"], bo, n1g, n1b, n2g, n2b,
          w11a, b11, w11c)

    def call_block2(he_c, g2_c, pre11_c):
        return pl.pallas_call(
            _block2_body,
            grid=grid,
            in_specs=[
                _row_spec(ne, Hd),
                _row_spec(ne, Hd),
                _row_spec(_NB, Hd),
                _const_spec(Hd, Hd),
                _const_spec(Hd, Hd),
                _const_spec(1, Hd),
                _const_spec(Hd, Hd),
                _const_spec(1, Hd),
                _const_spec(1, Hd),
                _const_spec(1, Hd),
            ],
            out_specs=[_row_spec(ne, Hd)],
            out_shape=[jax.ShapeDtypeStruct((echunk, Hd), jnp.float32)],
        )(he_c, g2_c, pre11_c, w11b, p["W12"]["w"], b12, p["W13"]["w"], b13,
          n3g, n3b)[0]

    g1s = [_sc_gather(g1t, idx_chunk(c))[:echunk] for c in range(_NCHUNKS)]
    b1outs = [
        call_block1(
            hv[c * nchunk:(c + 1) * nchunk],
            pre1[c * nchunk:(c + 1) * nchunk],
            he[c * echunk:(c + 1) * echunk],
            g1s[c],
        )
        for c in range(_NCHUNKS)
    ]
    hv2 = jnp.concatenate([o[0] for o in b1outs])
    pre11 = jnp.concatenate([o[1] for o in b1outs])
    g11t = jnp.concatenate([o[2] for o in b1outs])

    g2s = [_sc_gather(g11t, idx_chunk(c))[:echunk] for c in range(_NCHUNKS)]
    heos = [
        call_block2(
            he[c * echunk:(c + 1) * echunk],
            g2s[c],
            pre11[c * nchunk:(c + 1) * nchunk],
        )
        for c in range(_NCHUNKS)
    ]
    heo = jnp.concatenate(heos)

    return hv2.reshape(B, N, Hd), heo.reshape(B, N, K, Hd)


# speedup vs baseline: 1.3273x; 1.3273x over previous
import functools

import jax
import jax.numpy as jnp
from jax.experimental import pallas as pl
from jax.experimental.pallas import tpu as pltpu
from jax.experimental.pallas import tpu_sc as plsc

_GC = 128
_WORKERS = 32
_NB = 200


def _gelu(x):
    return x * 0.5 * (1.0 + jax.lax.erf(x * 0.7071067811865476))


def _ln(x, g, b):
    m = jnp.mean(x, axis=-1, keepdims=True)
    d = x - m
    v = jnp.mean(d * d, axis=-1, keepdims=True)
    return d * jax.lax.rsqrt(v + 1e-5) * g + b


def _sc_gather(table, idx1d):
    Mp = idx1d.shape[0]
    D = table.shape[1]
    idx2d = idx1d.reshape(1, Mp)
    mesh = plsc.VectorSubcoreMesh(core_axis_name="c", subcore_axis_name="s")

    @functools.partial(
        pl.kernel,
        out_type=jax.ShapeDtypeStruct((Mp, D), table.dtype),
        mesh=mesh,
    )
    def run(x_hbm, i_hbm, o_hbm):
        def body(i_vmem, o_vmem):
            pltpu.sync_copy(x_hbm.at[i_vmem.at[0]], o_vmem)

        pltpu.emit_pipeline(
            body,
            grid=(Mp // _GC,),
            in_specs=[pl.BlockSpec((1, _GC), lambda i: (0, i))],
            out_specs=[pl.BlockSpec((_GC, D), lambda i: (i, 0))],
            core_axis_name=("c", "s"),
            dimension_semantics=(pltpu.PARALLEL,),
        )(i_hbm, o_hbm)

    return run(table, idx2d)


def _proj_body(hv, w1a, b1, w1c, pre1_o, g1_o):
    x = hv[...]
    pre1_o[...] = (
        jnp.dot(x, w1a[...], preferred_element_type=jnp.float32) + b1[...]
    )
    g1_o[...] = jnp.dot(x, w1c[...], preferred_element_type=jnp.float32)


def _block1_body(hv, pre1, he, g1, w1b, w2, b2, w3, b3, win, bi, wout, bo,
                 n1g, n1b, n2g, n2b, w11a, b11, w11c,
                 hv2_o, pre11_o, g11_o):
    nb, Hd = hv.shape
    ne = he.shape[0]
    K = ne // nb
    e = jnp.dot(he[...], w1b[...], preferred_element_type=jnp.float32)
    t = (e + g1[...]).reshape(nb, K, Hd) + pre1[...][:, None, :]
    t = _gelu(t).reshape(ne, Hd)
    u = _gelu(jnp.dot(t, w2[...], preferred_element_type=jnp.float32) + b2[...])
    msg = jnp.dot(u, w3[...], preferred_element_type=jnp.float32) + b3[...]
    dh = msg.reshape(nb, K, Hd).sum(axis=1) * (1.0 / 30.0)
    h = _ln(hv[...] + dh, n1g[...], n1b[...])
    f = _gelu(jnp.dot(h, win[...], preferred_element_type=jnp.float32) + bi[...])
    f = jnp.dot(f, wout[...], preferred_element_type=jnp.float32) + bo[...]
    y = _ln(h + f, n2g[...], n2b[...])
    hv2_o[...] = y
    pre11_o[...] = (
        jnp.dot(y, w11a[...], preferred_element_type=jnp.float32) + b11[...]
    )
    g11_o[...] = jnp.dot(y, w11c[...], preferred_element_type=jnp.float32)


def _block2_body(he, g2, pre11, w11b, w12, b12, w13, b13, n3g, n3b, heo):
    ne, Hd = he.shape
    nb = pre11.shape[0]
    K = ne // nb
    e = jnp.dot(he[...], w11b[...], preferred_element_type=jnp.float32)
    t = (e + g2[...]).reshape(nb, K, Hd) + pre11[...][:, None, :]
    t = _gelu(t).reshape(ne, Hd)
    u = _gelu(jnp.dot(t, w12[...], preferred_element_type=jnp.float32) + b12[...])
    msg = jnp.dot(u, w13[...], preferred_element_type=jnp.float32) + b13[...]
    heo[...] = _ln(he[...] + msg, n3g[...], n3b[...])


def _row_spec(rows, cols):
    return pl.BlockSpec((rows, cols), lambda i: (i, 0))


def _const_spec(rows, cols):
    return pl.BlockSpec((rows, cols), lambda i: (0, 0))


def kernel(h_V, h_E, E_idx, params):
    B, N, Hd = h_V.shape
    K = h_E.shape[2]
    M = N * K
    hv = h_V.reshape(N, Hd)
    he = h_E.reshape(M, Hd)
    idx = E_idx.reshape(-1).astype(jnp.int32)

    p = params
    w1 = p["W1"]["w"]
    w11 = p["W11"]["w"]
    w1a, w1b, w1c = w1[:Hd], w1[Hd:2 * Hd], w1[2 * Hd:]
    w11a, w11b, w11c = w11[:Hd], w11[Hd:2 * Hd], w11[2 * Hd:]
    b1 = p["W1"]["b"].reshape(1, Hd)
    b11 = p["W11"]["b"].reshape(1, Hd)
    b2 = p["W2"]["b"].reshape(1, Hd)
    b3 = p["W3"]["b"].reshape(1, Hd)
    b12 = p["W12"]["b"].reshape(1, Hd)
    b13 = p["W13"]["b"].reshape(1, Hd)
    bi = p["Win"]["b"].reshape(1, -1)
    bo = p["Wout"]["b"].reshape(1, Hd)
    n1g = p["norm1"]["g"].reshape(1, Hd)
    n1b = p["norm1"]["b"].reshape(1, Hd)
    n2g = p["norm2"]["g"].reshape(1, Hd)
    n2b = p["norm2"]["b"].reshape(1, Hd)
    n3g = p["norm3"]["g"].reshape(1, Hd)
    n3b = p["norm3"]["b"].reshape(1, Hd)
    Hi = p["Win"]["w"].shape[1]

    align = _GC * _WORKERS
    Mp = ((M + align - 1) // align) * align
    if Mp != M:
        idx = jnp.concatenate([idx, jnp.zeros((Mp - M,), jnp.int32)])

    pre1, g1t = pl.pallas_call(
        _proj_body,
        out_shape=[
            jax.ShapeDtypeStruct((N, Hd), jnp.float32),
            jax.ShapeDtypeStruct((N, Hd), jnp.float32),
        ],
    )(hv, w1a, b1, w1c)

    grid = (N // _NB,)
    ne = _NB * K

    g1 = _sc_gather(g1t, idx)[:M]

    hv2, pre11, g11t = pl.pallas_call(
        _block1_body,
        grid=grid,
        in_specs=[
            _row_spec(_NB, Hd),
            _row_spec(_NB, Hd),
            _row_spec(ne, Hd),
            _row_spec(ne, Hd),
            _const_spec(Hd, Hd),
            _const_spec(Hd, Hd),
            _const_spec(1, Hd),
            _const_spec(Hd, Hd),
            _const_spec(1, Hd),
            _const_spec(Hd, Hi),
            _const_spec(1, Hi),
            _const_spec(Hi, Hd),
            _const_spec(1, Hd),
            _const_spec(1, Hd),
            _const_spec(1, Hd),
            _const_spec(1, Hd),
            _const_spec(1, Hd),
            _const_spec(Hd, Hd),
            _const_spec(1, Hd),
            _const_spec(Hd, Hd),
        ],
        out_specs=[
            _row_spec(_NB, Hd),
            _row_spec(_NB, Hd),
            _row_spec(_NB, Hd),
        ],
        out_shape=[
            jax.ShapeDtypeStruct((N, Hd), jnp.float32),
            jax.ShapeDtypeStruct((N, Hd), jnp.float32),
            jax.ShapeDtypeStruct((N, Hd), jnp.float32),
        ],
    )(hv, pre1, he, g1, w1b, p["W2"]["w"], b2, p["W3"]["w"], b3,
      p["Win"]["w"], bi, p["Wout"]["w"], bo, n1g, n1b, n2g, n2b,
      w11a, b11, w11c)

    g2 = _sc_gather(g11t, idx)[:M]

    heo = pl.pallas_call(
        _block2_body,
        grid=grid,
        in_specs=[
            _row_spec(ne, Hd),
            _row_spec(ne, Hd),
            _row_spec(_NB, Hd),
            _const_spec(Hd, Hd),
            _const_spec(Hd, Hd),
            _const_spec(1, Hd),
            _const_spec(Hd, Hd),
            _const_spec(1, Hd),
            _const_spec(1, Hd),
            _const_spec(1, Hd),
        ],
        out_specs=[_row_spec(ne, Hd)],
        out_shape=[jax.ShapeDtypeStruct((M, Hd), jnp.float32)],
    )(he, g2, pre11, w11b, p["W12"]["w"], b12, p["W13"]["w"], b13,
      n3g, n3b)[0]

    return hv2.reshape(B, N, Hd), heo.reshape(B, N, K, Hd)


# speedup vs baseline: 1.3292x; 1.0015x over previous
import functools

import jax
import jax.numpy as jnp
from jax.experimental import pallas as pl
from jax.experimental.pallas import tpu as pltpu
from jax.experimental.pallas import tpu_sc as plsc

_GC = 128
_WORKERS = 32
_NB = 200


def _gelu(x):
    return x * 0.5 * (1.0 + jax.lax.erf(x * 0.7071067811865476))


def _ln(x, g, b):
    m = jnp.mean(x, axis=-1, keepdims=True)
    d = x - m
    v = jnp.mean(d * d, axis=-1, keepdims=True)
    return d * jax.lax.rsqrt(v + 1e-5) * g + b


def _sc_gather(table, idx1d):
    Mp = idx1d.shape[0]
    D = table.shape[1]
    idx2d = idx1d.reshape(1, Mp)
    mesh = plsc.VectorSubcoreMesh(core_axis_name="c", subcore_axis_name="s")

    @functools.partial(
        pl.kernel,
        out_type=jax.ShapeDtypeStruct((Mp, D), table.dtype),
        mesh=mesh,
    )
    def run(x_hbm, i_hbm, o_hbm):
        def body(i_vmem, o_vmem):
            pltpu.sync_copy(x_hbm.at[i_vmem.at[0]], o_vmem)

        pltpu.emit_pipeline(
            body,
            grid=(Mp // _GC,),
            in_specs=[pl.BlockSpec((1, _GC), lambda i: (0, i))],
            out_specs=[pl.BlockSpec((_GC, D), lambda i: (i, 0))],
            core_axis_name=("c", "s"),
            dimension_semantics=(pltpu.PARALLEL,),
        )(i_hbm, o_hbm)

    return run(table, idx2d)


def _bdot(a, b):
    return jnp.dot(a.astype(jnp.bfloat16), b.astype(jnp.bfloat16),
                   preferred_element_type=jnp.float32)


def _proj_body(hv, w1a, b1, w1c, pre1_o, g1_o):
    x = hv[...]
    pre1_o[...] = (
        jnp.dot(x, w1a[...], preferred_element_type=jnp.float32) + b1[...]
    )
    g1_o[...] = jnp.dot(x, w1c[...], preferred_element_type=jnp.float32)


def _block1_body(hv, pre1, he, g1, w1b, w2, b2, w3, b3, win, bi, wout, bo,
                 n1g, n1b, n2g, n2b, w11a, b11, w11c,
                 hv2_o, pre11_o, g11_o):
    nb, Hd = hv.shape
    ne = he.shape[0]
    K = ne // nb
    e = _bdot(he[...], w1b[...])
    t = (e + g1[...].astype(jnp.float32)).reshape(nb, K, Hd) + pre1[...][:, None, :]
    t = _gelu(t).reshape(ne, Hd)
    u = _gelu(_bdot(t, w2[...]) + b2[...])
    msg = _bdot(u, w3[...]) + b3[...]
    dh = msg.reshape(nb, K, Hd).sum(axis=1) * (1.0 / 30.0)
    h = _ln(hv[...] + dh, n1g[...], n1b[...])
    f = _gelu(jnp.dot(h, win[...], preferred_element_type=jnp.float32) + bi[...])
    f = jnp.dot(f, wout[...], preferred_element_type=jnp.float32) + bo[...]
    y = _ln(h + f, n2g[...], n2b[...])
    hv2_o[...] = y
    pre11_o[...] = (
        jnp.dot(y, w11a[...], preferred_element_type=jnp.float32) + b11[...]
    )
    g11_o[...] = jnp.dot(y, w11c[...], preferred_element_type=jnp.float32)


def _block2_body(he, g2, pre11, w11b, w12, b12, w13, b13, n3g, n3b, heo):
    ne, Hd = he.shape
    nb = pre11.shape[0]
    K = ne // nb
    e = _bdot(he[...], w11b[...])
    t = (e + g2[...].astype(jnp.float32)).reshape(nb, K, Hd) + pre11[...][:, None, :]
    t = _gelu(t).reshape(ne, Hd)
    u = _gelu(_bdot(t, w12[...]) + b12[...])
    msg = _bdot(u, w13[...]) + b13[...]
    heo[...] = _ln(he[...] + msg, n3g[...], n3b[...])


def _row_spec(rows, cols):
    return pl.BlockSpec((rows, cols), lambda i: (i, 0))


def _const_spec(rows, cols):
    return pl.BlockSpec((rows, cols), lambda i: (0, 0))


def kernel(h_V, h_E, E_idx, params):
    B, N, Hd = h_V.shape
    K = h_E.shape[2]
    M = N * K
    hv = h_V.reshape(N, Hd)
    he = h_E.reshape(M, Hd)
    idx = E_idx.reshape(-1).astype(jnp.int32)

    p = params
    w1 = p["W1"]["w"]
    w11 = p["W11"]["w"]
    w1a, w1b, w1c = w1[:Hd], w1[Hd:2 * Hd], w1[2 * Hd:]
    w11a, w11b, w11c = w11[:Hd], w11[Hd:2 * Hd], w11[2 * Hd:]
    b1 = p["W1"]["b"].reshape(1, Hd)
    b11 = p["W11"]["b"].reshape(1, Hd)
    b2 = p["W2"]["b"].reshape(1, Hd)
    b3 = p["W3"]["b"].reshape(1, Hd)
    b12 = p["W12"]["b"].reshape(1, Hd)
    b13 = p["W13"]["b"].reshape(1, Hd)
    bi = p["Win"]["b"].reshape(1, -1)
    bo = p["Wout"]["b"].reshape(1, Hd)
    n1g = p["norm1"]["g"].reshape(1, Hd)
    n1b = p["norm1"]["b"].reshape(1, Hd)
    n2g = p["norm2"]["g"].reshape(1, Hd)
    n2b = p["norm2"]["b"].reshape(1, Hd)
    n3g = p["norm3"]["g"].reshape(1, Hd)
    n3b = p["norm3"]["b"].reshape(1, Hd)
    Hi = p["Win"]["w"].shape[1]

    align = _GC * _WORKERS
    Mp = ((M + align - 1) // align) * align
    if Mp != M:
        idx = jnp.concatenate([idx, jnp.zeros((Mp - M,), jnp.int32)])

    pre1, g1t = pl.pallas_call(
        _proj_body,
        out_shape=[
            jax.ShapeDtypeStruct((N, Hd), jnp.float32),
            jax.ShapeDtypeStruct((N, Hd), jnp.float32),
        ],
    )(hv, w1a, b1, w1c)

    grid = (N // _NB,)
    ne = _NB * K

    g1 = _sc_gather(g1t, idx)[:M]

    hv2, pre11, g11t = pl.pallas_call(
        _block1_body,
        grid=grid,
        in_specs=[
            _row_spec(_NB, Hd),
            _row_spec(_NB, Hd),
            _row_spec(ne, Hd),
            _row_spec(ne, Hd),
            _const_spec(Hd, Hd),
            _const_spec(Hd, Hd),
            _const_spec(1, Hd),
            _const_spec(Hd, Hd),
            _const_spec(1, Hd),
            _const_spec(Hd, Hi),
            _const_spec(1, Hi),
            _const_spec(Hi, Hd),
            _const_spec(1, Hd),
            _const_spec(1, Hd),
            _const_spec(1, Hd),
            _const_spec(1, Hd),
            _const_spec(1, Hd),
            _const_spec(Hd, Hd),
            _const_spec(1, Hd),
            _const_spec(Hd, Hd),
        ],
        out_specs=[
            _row_spec(_NB, Hd),
            _row_spec(_NB, Hd),
            _row_spec(_NB, Hd),
        ],
        out_shape=[
            jax.ShapeDtypeStruct((N, Hd), jnp.float32),
            jax.ShapeDtypeStruct((N, Hd), jnp.float32),
            jax.ShapeDtypeStruct((N, Hd), jnp.float32),
        ],
    )(hv, pre1, he, g1, w1b, p["W2"]["w"], b2, p["W3"]["w"], b3,
      p["Win"]["w"], bi, p["Wout"]["w"], bo, n1g, n1b, n2g, n2b,
      w11a, b11, w11c)

    g2 = _sc_gather(g11t, idx)[:M]

    heo = pl.pallas_call(
        _block2_body,
        grid=grid,
        in_specs=[
            _row_spec(ne, Hd),
            _row_spec(ne, Hd),
            _row_spec(_NB, Hd),
            _const_spec(Hd, Hd),
            _const_spec(Hd, Hd),
            _const_spec(1, Hd),
            _const_spec(Hd, Hd),
            _const_spec(1, Hd),
            _const_spec(1, Hd),
            _const_spec(1, Hd),
        ],
        out_specs=[_row_spec(ne, Hd)],
        out_shape=[jax.ShapeDtypeStruct((M, Hd), jnp.float32)],
    )(he, g2, pre11, w11b, p["W12"]["w"], b12, p["W13"]["w"], b13,
      n3g, n3b)[0]

    return hv2.reshape(B, N, Hd), heo.reshape(B, N, K, Hd)
